# baseline (speedup 1.0000x reference)
import numpy as np
import jax
import jax.numpy as jnp
from jax import lax
from jax.experimental import pallas as pl
from jax.experimental.pallas import tpu as pltpu
from jax.experimental.pallas import tpu_sc as plsc

B = 16384
K = 32
N_ROWS = 1000000
D = 16
H = 20
NC, NS = 2, 16
NW = NC * NS
BPW = B // NW
CPW = BPW // 128
APW = BPW * H // 128
G = 8
GC = APW // G
GR = GC * 128

_j_over_h = (np.arange(BPW * H) // H).astype(np.int32)
_SIDX_ALL = np.stack(
    [(w // NC) * BPW + _j_over_h for w in range(NW)]
).reshape(NW, APW, 128)


def _sc_body(u_ref, i_ref, uo_ref, io_ref, ia_ref, sidx_ref, P_ref, Q_ref,
             bu_ref, bi_ref, ae_ref, uf_out, if_out, as_out, buv_out, biv_out,
             iu_v, ii_v, iuo_v, iio_v, ia_v, sx_v, uf_v, if_v, bu_v, bi_v,
             rows_v, acc_s, sem, sem2, sg0, sg1, se0, se1):
    c = lax.axis_index("c")
    s = lax.axis_index("s")
    wid = s * NC + c
    pltpu.sync_copy(u_ref.at[wid], iu_v)
    pltpu.sync_copy(i_ref.at[wid], ii_v)
    pltpu.sync_copy(uo_ref.at[wid], iuo_v)
    pltpu.sync_copy(io_ref.at[wid], iio_v)
    pltpu.sync_copy(ia_ref.at[wid], ia_v)
    pltpu.sync_copy(sidx_ref.at[wid], sx_v)

    handles = []
    for cc in range(CPW):
        dst = pl.ds(cc * 128, 128)
        handles.append(pltpu.async_copy(bu_ref.at[iuo_v.at[cc]], bu_v.at[dst], sem))
        handles.append(pltpu.async_copy(bi_ref.at[iio_v.at[cc]], bi_v.at[dst], sem))

    sg = (sg0, sg1)
    se = (se0, se1)
    gh = [None] * CPW
    eh = [None] * CPW
    for cc in range(2):
        gh[cc] = (pltpu.async_copy(P_ref.at[iu_v.at[cc]], uf_v.at[cc], sg[cc]),
                  pltpu.async_copy(Q_ref.at[ii_v.at[cc]], if_v.at[cc], sg[cc]))
    for cc in range(CPW):
        slot = cc % 2
        gh[cc][0].wait()
        gh[cc][1].wait()
        dst = pl.ds(wid * BPW + cc * 128, 128)
        eh[cc] = (pltpu.async_copy(uf_v.at[slot], uf_out.at[dst], se[slot]),
                  pltpu.async_copy(if_v.at[slot], if_out.at[dst], se[slot]))
        nxt = cc + 2
        if nxt < CPW:
            eh[cc][0].wait()
            eh[cc][1].wait()
            gh[nxt] = (pltpu.async_copy(P_ref.at[iu_v.at[nxt]], uf_v.at[slot], sg[slot]),
                       pltpu.async_copy(Q_ref.at[ii_v.at[nxt]], if_v.at[slot], sg[slot]))
    for cc in range(CPW - 2, CPW):
        eh[cc][0].wait()
        eh[cc][1].wait()

    def zero_body(j, carry):
        rows_v[j] = jnp.zeros((D,), jnp.float32)
        return carry
    lax.fori_loop(0, BPW, zero_body, 0)
    pltpu.sync_copy(rows_v.at[pl.ds(0, BPW)], acc_s.at[pl.ds(s * BPW, BPW)])

    def group_body(g, carry):
        gh = []
        for j in range(GC):
            gh.append(pltpu.async_copy(
                ae_ref.at[ia_v.at[g * GC + j]],
                rows_v.at[pl.ds(j * 128, 128)], sem2))
        for hnd in gh:
            hnd.wait()
        for j in range(GC):
            pltpu.sync_copy(rows_v.at[pl.ds(j * 128, 128)],
                            acc_s.at[sx_v.at[g * GC + j]], add=True)
        return carry
    lax.fori_loop(0, G, group_body, 0)

    for hnd in handles:
        hnd.wait()

    out = pl.ds(wid * BPW, BPW)
    pltpu.sync_copy(bu_v, buv_out.at[out])
    pltpu.sync_copy(bi_v, biv_out.at[out])
    pltpu.sync_copy(acc_s.at[pl.ds(s * BPW, BPW)], as_out.at[out])


def _tc_body(uf4_ref, if4_ref, uq_ref, iq_ref, as_ref, buv_ref, biv_ref,
             wt_ref, bf_ref, mu_ref, out_ref):
    uf4 = uf4_ref[...]
    if4 = if4_ref[...]
    uq = uq_ref[...][:, None]
    iq = iq_ref[...][:, None]
    uf = jnp.zeros(uf4[:, :K].shape, jnp.float32)
    itf = jnp.zeros_like(uf)
    for q in range(4):
        uf = uf + jnp.where(uq == q, uf4[:, q * K:(q + 1) * K], 0.0)
        itf = itf + jnp.where(iq == q, if4[:, q * K:(q + 1) * K], 0.0)
    avg = as_ref[...] * (1.0 / H)
    attr = jnp.dot(avg, wt_ref[...], preferred_element_type=jnp.float32)
    itf = itf + attr + bf_ref[...]
    pred = jnp.sum(uf * itf, axis=1)
    out_ref[...] = pred + buv_ref[...] + biv_ref[...] + mu_ref[0, 0]


def _make_sc_call():
    f32 = jnp.float32
    return pl.kernel(
        _sc_body,
        out_type=[
            jax.ShapeDtypeStruct((B, 128), f32),
            jax.ShapeDtypeStruct((B, 128), f32),
            jax.ShapeDtypeStruct((B, D), f32),
            jax.ShapeDtypeStruct((B,), f32),
            jax.ShapeDtypeStruct((B,), f32),
        ],
        mesh=plsc.VectorSubcoreMesh(core_axis_name="c", subcore_axis_name="s"),
        scratch_types=[
            pltpu.VMEM((CPW, 128), jnp.int32),
            pltpu.VMEM((CPW, 128), jnp.int32),
            pltpu.VMEM((CPW, 128), jnp.int32),
            pltpu.VMEM((CPW, 128), jnp.int32),
            pltpu.VMEM((APW, 128), jnp.int32),
            pltpu.VMEM((APW, 128), jnp.int32),
            pltpu.VMEM((2, 128, 128), f32),
            pltpu.VMEM((2, 128, 128), f32),
            pltpu.VMEM((BPW,), f32),
            pltpu.VMEM((BPW,), f32),
            pltpu.VMEM((GR, D), f32),
            pltpu.VMEM_SHARED((NS * BPW, D), f32),
            pltpu.SemaphoreType.DMA,
            pltpu.SemaphoreType.DMA,
            pltpu.SemaphoreType.DMA,
            pltpu.SemaphoreType.DMA,
            pltpu.SemaphoreType.DMA,
            pltpu.SemaphoreType.DMA,
        ],
        compiler_params=pltpu.CompilerParams(use_tc_tiling_on_sc=False),
    )


def kernel(u, i, item_attrs, P, Q, bu, bi, mu, attr_emb, W_fusion, b_fusion):
    ui = u.astype(jnp.int32)
    ii = i.astype(jnp.int32)
    u3 = (ui // 4).reshape(NW, CPW, 128)
    i3 = (ii // 4).reshape(NW, CPW, 128)
    u3o = ui.reshape(NW, CPW, 128)
    i3o = ii.reshape(NW, CPW, 128)
    ia3 = item_attrs.astype(jnp.int32).reshape(NW, APW, 128)
    sidx = jnp.asarray(_SIDX_ALL)

    f32 = jnp.float32
    P4 = P.reshape(N_ROWS // 4, 128)
    Q4 = Q.reshape(N_ROWS // 4, 128)
    uf4, if4, asum, buv, biv = _make_sc_call()(
        u3, i3, u3o, i3o, ia3, sidx, P4, Q4,
        bu.reshape(-1), bi.reshape(-1), attr_emb)

    TB = 2048
    combine = pl.pallas_call(
        _tc_body,
        grid=(B // TB,),
        in_specs=[
            pl.BlockSpec((TB, 128), lambda j: (j, 0)),
            pl.BlockSpec((TB, 128), lambda j: (j, 0)),
            pl.BlockSpec((TB,), lambda j: (j,)),
            pl.BlockSpec((TB,), lambda j: (j,)),
            pl.BlockSpec((TB, D), lambda j: (j, 0)),
            pl.BlockSpec((TB,), lambda j: (j,)),
            pl.BlockSpec((TB,), lambda j: (j,)),
            pl.BlockSpec((D, K), lambda j: (0, 0)),
            pl.BlockSpec((1, K), lambda j: (0, 0)),
            pl.BlockSpec((1, 1), lambda j: (0, 0)),
        ],
        out_specs=pl.BlockSpec((TB,), lambda j: (j,)),
        out_shape=jax.ShapeDtypeStruct((B,), f32),
    )
    return combine(uf4, if4, ui % 4, ii % 4, asum, buv, biv,
                   W_fusion.T.astype(f32), b_fusion.reshape(1, K),
                   mu.reshape(1, 1))


# speedup vs baseline: 1.3661x; 1.3661x over previous
import numpy as np
import jax
import jax.numpy as jnp
from jax import lax
from jax.experimental import pallas as pl
from jax.experimental.pallas import tpu as pltpu
from jax.experimental.pallas import tpu_sc as plsc

B = 16384
K = 32
N_ROWS = 1000000
D = 16
H = 20
NC, NS = 2, 16
NW = NC * NS
BPW = B // NW
CPW = BPW // 128
APW = BPW * H // 128
G = 8
GC = APW // G
GR = GC * 128

_j_over_h = (np.arange(BPW * H) // H).astype(np.int32)
_SIDX_ALL = np.stack(
    [(w // NC) * BPW + _j_over_h for w in range(NW)]
).reshape(NW, APW, 128)


def _sc_body(u_ref, i_ref, uo_ref, io_ref, ia_ref, sidx_ref, P_ref, Q_ref,
             bu_ref, bi_ref, ae_ref, uf_out, if_out, as_out, buv_out, biv_out,
             iu_v, ii_v, iuo_v, iio_v, ia_v, sx_v, uf_v, if_v, bu_v, bi_v,
             rows_v, acc_s, sem, sem2, sg0, sg1, se0, se1):
    c = lax.axis_index("c")
    s = lax.axis_index("s")
    wid = s * NC + c
    pltpu.sync_copy(u_ref.at[wid], iu_v)
    pltpu.sync_copy(i_ref.at[wid], ii_v)
    pltpu.sync_copy(uo_ref.at[wid], iuo_v)
    pltpu.sync_copy(io_ref.at[wid], iio_v)
    pltpu.sync_copy(ia_ref.at[wid], ia_v)
    pltpu.sync_copy(sidx_ref.at[wid], sx_v)

    handles = []
    for cc in range(CPW):
        dst = pl.ds(cc * 128, 128)
        handles.append(pltpu.async_copy(bu_ref.at[iuo_v.at[cc]], bu_v.at[dst], sem))
        handles.append(pltpu.async_copy(bi_ref.at[iio_v.at[cc]], bi_v.at[dst], sem))

    sg = (sg0, sg1)
    se = (se0, se1)
    gh = [None] * CPW
    eh = [None] * CPW
    for cc in range(2):
        gh[cc] = (pltpu.async_copy(P_ref.at[iu_v.at[cc]], uf_v.at[cc], sg[cc]),
                  pltpu.async_copy(Q_ref.at[ii_v.at[cc]], if_v.at[cc], sg[cc]))
    for cc in range(CPW):
        slot = cc % 2
        gh[cc][0].wait()
        gh[cc][1].wait()
        dst = pl.ds(wid * BPW + cc * 128, 128)
        eh[cc] = (pltpu.async_copy(uf_v.at[slot], uf_out.at[dst], se[slot]),
                  pltpu.async_copy(if_v.at[slot], if_out.at[dst], se[slot]))
        nxt = cc + 2
        if nxt < CPW:
            eh[cc][0].wait()
            eh[cc][1].wait()
            gh[nxt] = (pltpu.async_copy(P_ref.at[iu_v.at[nxt]], uf_v.at[slot], sg[slot]),
                       pltpu.async_copy(Q_ref.at[ii_v.at[nxt]], if_v.at[slot], sg[slot]))
    for cc in range(CPW - 2, CPW):
        eh[cc][0].wait()
        eh[cc][1].wait()

    def zero_body(j, carry):
        rows_v[j] = jnp.zeros((D,), jnp.float32)
        return carry
    lax.fori_loop(0, BPW, zero_body, 0)
    pltpu.sync_copy(rows_v.at[pl.ds(0, BPW)], acc_s.at[pl.ds(s * BPW, BPW)])

    def group_body(g, carry):
        gh = []
        for j in range(GC):
            gh.append(pltpu.async_copy(
                ae_ref.at[ia_v.at[g * GC + j]],
                rows_v.at[pl.ds(j * 128, 128)], sem2))
        for hnd in gh:
            hnd.wait()
        for j in range(GC):
            pltpu.sync_copy(rows_v.at[pl.ds(j * 128, 128)],
                            acc_s.at[sx_v.at[g * GC + j]], add=True)
        return carry
    lax.fori_loop(0, G, group_body, 0)

    for hnd in handles:
        hnd.wait()

    out = pl.ds(wid * BPW, BPW)
    pltpu.sync_copy(bu_v, buv_out.at[out])
    pltpu.sync_copy(bi_v, biv_out.at[out])
    pltpu.sync_copy(acc_s.at[pl.ds(s * BPW, BPW)], as_out.at[out])


PACK_W = 32768
PACK_S = PACK_W // 4
PACK_G = -(-N_ROWS // PACK_W)
PACK_ROWS = PACK_G * PACK_S


def _pack_body(x_ref, out_ref):
    eye = (lax.broadcasted_iota(jnp.int32, (K, K), 0)
           == lax.broadcasted_iota(jnp.int32, (K, K), 1)).astype(jnp.bfloat16)
    for g in range(4):
        x = x_ref[:, g * PACK_S:(g + 1) * PACK_S]
        hi = x.astype(jnp.bfloat16)
        lo = (x - hi.astype(jnp.float32)).astype(jnp.bfloat16)
        z = (lax.dot_general(hi, eye, (((0,), (0,)), ((), ())),
                             preferred_element_type=jnp.float32)
             + lax.dot_general(lo, eye, (((0,), (0,)), ((), ())),
                               preferred_element_type=jnp.float32))
        out_ref[:, g * K:(g + 1) * K] = z


def _pack4(xt):
    return pl.pallas_call(
        _pack_body,
        grid=(PACK_G,),
        in_specs=[pl.BlockSpec((K, PACK_W), lambda j: (0, j))],
        out_specs=pl.BlockSpec((PACK_S, 128), lambda j: (j, 0)),
        out_shape=jax.ShapeDtypeStruct((PACK_ROWS, 128), jnp.float32),
    )(xt)


def _tc_body(uf4_ref, if4_ref, uq_ref, iq_ref, as_ref, buv_ref, biv_ref,
             wt_ref, bf_ref, mu_ref, out_ref):
    uf4 = uf4_ref[...]
    if4 = if4_ref[...]
    uq = uq_ref[...][:, None]
    iq = iq_ref[...][:, None]
    uf = jnp.zeros(uf4[:, :K].shape, jnp.float32)
    itf = jnp.zeros_like(uf)
    for q in range(4):
        uf = uf + jnp.where(uq == q, uf4[:, q * K:(q + 1) * K], 0.0)
        itf = itf + jnp.where(iq == q, if4[:, q * K:(q + 1) * K], 0.0)
    avg = as_ref[...] * (1.0 / H)
    attr = jnp.dot(avg, wt_ref[...], preferred_element_type=jnp.float32)
    itf = itf + attr + bf_ref[...]
    pred = jnp.sum(uf * itf, axis=1)
    out_ref[...] = pred + buv_ref[...] + biv_ref[...] + mu_ref[0, 0]


def _make_sc_call():
    f32 = jnp.float32
    return pl.kernel(
        _sc_body,
        out_type=[
            jax.ShapeDtypeStruct((B, 128), f32),
            jax.ShapeDtypeStruct((B, 128), f32),
            jax.ShapeDtypeStruct((B, D), f32),
            jax.ShapeDtypeStruct((B,), f32),
            jax.ShapeDtypeStruct((B,), f32),
        ],
        mesh=plsc.VectorSubcoreMesh(core_axis_name="c", subcore_axis_name="s"),
        scratch_types=[
            pltpu.VMEM((CPW, 128), jnp.int32),
            pltpu.VMEM((CPW, 128), jnp.int32),
            pltpu.VMEM((CPW, 128), jnp.int32),
            pltpu.VMEM((CPW, 128), jnp.int32),
            pltpu.VMEM((APW, 128), jnp.int32),
            pltpu.VMEM((APW, 128), jnp.int32),
            pltpu.VMEM((2, 128, 128), f32),
            pltpu.VMEM((2, 128, 128), f32),
            pltpu.VMEM((BPW,), f32),
            pltpu.VMEM((BPW,), f32),
            pltpu.VMEM((GR, D), f32),
            pltpu.VMEM_SHARED((NS * BPW, D), f32),
            pltpu.SemaphoreType.DMA,
            pltpu.SemaphoreType.DMA,
            pltpu.SemaphoreType.DMA,
            pltpu.SemaphoreType.DMA,
            pltpu.SemaphoreType.DMA,
            pltpu.SemaphoreType.DMA,
        ],
        compiler_params=pltpu.CompilerParams(use_tc_tiling_on_sc=False),
    )


def kernel(u, i, item_attrs, P, Q, bu, bi, mu, attr_emb, W_fusion, b_fusion):
    ui = u.astype(jnp.int32)
    ii = i.astype(jnp.int32)
    u3 = (((ui >> 15) << 13) + (ui & (PACK_S - 1))).reshape(NW, CPW, 128)
    i3 = (((ii >> 15) << 13) + (ii & (PACK_S - 1))).reshape(NW, CPW, 128)
    u3o = ui.reshape(NW, CPW, 128)
    i3o = ii.reshape(NW, CPW, 128)
    ia3 = item_attrs.astype(jnp.int32).reshape(NW, APW, 128)
    sidx = jnp.asarray(_SIDX_ALL)

    f32 = jnp.float32
    P4 = _pack4(P.T)
    Q4 = _pack4(Q.T)
    uf4, if4, asum, buv, biv = _make_sc_call()(
        u3, i3, u3o, i3o, ia3, sidx, P4, Q4,
        bu.reshape(-1), bi.reshape(-1), attr_emb)

    TB = 2048
    combine = pl.pallas_call(
        _tc_body,
        grid=(B // TB,),
        in_specs=[
            pl.BlockSpec((TB, 128), lambda j: (j, 0)),
            pl.BlockSpec((TB, 128), lambda j: (j, 0)),
            pl.BlockSpec((TB,), lambda j: (j,)),
            pl.BlockSpec((TB,), lambda j: (j,)),
            pl.BlockSpec((TB, D), lambda j: (j, 0)),
            pl.BlockSpec((TB,), lambda j: (j,)),
            pl.BlockSpec((TB,), lambda j: (j,)),
            pl.BlockSpec((D, K), lambda j: (0, 0)),
            pl.BlockSpec((1, K), lambda j: (0, 0)),
            pl.BlockSpec((1, 1), lambda j: (0, 0)),
        ],
        out_specs=pl.BlockSpec((TB,), lambda j: (j,)),
        out_shape=jax.ShapeDtypeStruct((B,), f32),
    )
    return combine(uf4, if4, (ui >> 13) & 3, (ii >> 13) & 3, asum, buv, biv,
                   W_fusion.T.astype(f32), b_fusion.reshape(1, K),
                   mu.reshape(1, 1))
